# baseline (speedup 1.0000x reference)
import functools

import jax
import jax.numpy as jnp
from jax import lax
from jax.experimental import pallas as pl
from jax.experimental.pallas import tpu as pltpu
from jax.experimental.pallas import tpu_sc as plsc

VOCAB = 100000
EMBED_DIM = 64
BATCH = 4096
HIST = 50

NC = 2
NS = 16
NW = NC * NS
BPW = BATCH // NW
PAIRS = VOCAB // 2
PITCH = 136

_mesh = plsc.VectorSubcoreMesh(core_axis_name="c", subcore_axis_name="s")


@functools.partial(
    pl.kernel,
    mesh=_mesh,
    compiler_params=pltpu.CompilerParams(
        use_tc_tiling_on_sc=True, needs_layout_passes=False),
    out_type=jax.ShapeDtypeStruct((HIST, EMBED_DIM, BATCH), jnp.float32),
    scratch_types=[
        pltpu.VMEM((HIST, BPW), jnp.int32),
        pltpu.VMEM((BPW,), jnp.int32),
        pltpu.VMEM((BPW,), jnp.int32),
        pltpu.VMEM((BPW, PITCH), jnp.float32),
        pltpu.VMEM((BPW, PITCH), jnp.float32),
        pltpu.VMEM((EMBED_DIM, PITCH), jnp.float32),
        pltpu.VMEM((EMBED_DIM, PITCH), jnp.float32),
        pltpu.SemaphoreType.DMA,
        pltpu.SemaphoreType.DMA,
        pltpu.SemaphoreType.DMA,
        pltpu.SemaphoreType.DMA,
    ],
)
def _gather_kernel(table_p, idx_t, out, idx_v, p0, p1, gb0, gb1, ob0, ob1,
                   g0, g1, o0, o1):
    wid = lax.axis_index("s") * NC + lax.axis_index("c")
    b0 = wid * BPW
    pltpu.sync_copy(idx_t.at[:, pl.ds(b0, BPW)], idx_v)

    iota = lax.iota(jnp.int32, 16)
    rowm = [(iota + 16 * kb) * PITCH for kb in range(8)]

    def prep(h, pidx):
        for kb in range(8):
            x = idx_v[h, pl.ds(kb * 16, 16)]
            pidx[pl.ds(kb * 16, 16)] = x >> 1

    def gather(pidx, gb, sem):
        pltpu.async_copy(table_p.at[pidx], gb.at[:, pl.ds(0, 128)], sem)

    def wait_gather(pidx, gb, sem):
        pltpu.make_async_copy(table_p.at[pidx], gb.at[:, pl.ds(0, 128)], sem).wait()

    def repack(h, gb, ob):
        def kbody(kb, carry):
            xv = idx_v[h, pl.ds(kb * 16, 16)]
            offv = (xv & 1) << 6
            for kk in range(16):
                row = kb * 16 + kk
                off = offv[kk]
                kv = jnp.full((16,), row, jnp.int32)
                for jb in range(4):
                    x = gb[row, pl.ds(off + jb * 16, 16)]
                    plsc.store_scatter(ob, [iota + jb * 16, kv], x)
            return carry

        lax.fori_loop(0, 8, kbody, 0)

    def out_issue(h, ob, sem):
        pltpu.async_copy(ob.at[:, pl.ds(0, BPW)], out.at[h, :, pl.ds(b0, BPW)], sem)

    def wait_out(h, ob, sem):
        pltpu.make_async_copy(
            ob.at[:, pl.ds(0, BPW)], out.at[h, :, pl.ds(b0, BPW)], sem).wait()

    prep(0, p0)
    gather(p0, gb0, g0)
    prep(1, p1)
    gather(p1, gb1, g1)

    wait_gather(p0, gb0, g0)
    repack(0, gb0, ob0)
    out_issue(0, ob0, o0)
    prep(2, p0)
    gather(p0, gb0, g0)
    wait_gather(p1, gb1, g1)
    repack(1, gb1, ob1)
    out_issue(1, ob1, o1)
    prep(3, p1)
    gather(p1, gb1, g1)

    def body(t, carry):
        h0 = 2 * t
        h1 = 2 * t + 1
        wait_gather(p0, gb0, g0)
        wait_out(h0 - 2, ob0, o0)
        repack(h0, gb0, ob0)
        out_issue(h0, ob0, o0)
        prep(h0 + 2, p0)
        gather(p0, gb0, g0)
        wait_gather(p1, gb1, g1)
        wait_out(h1 - 2, ob1, o1)
        repack(h1, gb1, ob1)
        out_issue(h1, ob1, o1)
        prep(h1 + 2, p1)
        gather(p1, gb1, g1)
        return carry

    lax.fori_loop(1, HIST // 2 - 1, body, 0)

    wait_gather(p0, gb0, g0)
    wait_out(46, ob0, o0)
    repack(48, gb0, ob0)
    out_issue(48, ob0, o0)
    wait_gather(p1, gb1, g1)
    wait_out(47, ob1, o1)
    repack(49, gb1, ob1)
    out_issue(49, ob1, o1)
    wait_out(48, ob0, o0)
    wait_out(49, ob1, o1)


def kernel(indices, table):
    idx_t = indices.T
    table_p = table.reshape(PAIRS, 128)
    out_phys = _gather_kernel(table_p, idx_t)
    return jnp.transpose(out_phys, (2, 0, 1))


# speedup vs baseline: 1.0318x; 1.0318x over previous
import functools

import jax
import jax.numpy as jnp
from jax import lax
from jax.experimental import pallas as pl
from jax.experimental.pallas import tpu as pltpu
from jax.experimental.pallas import tpu_sc as plsc

VOCAB = 100000
EMBED_DIM = 64
BATCH = 4096
HIST = 50

NC = 2
NS = 16
NW = NC * NS
BPW = BATCH // NW
PAIRS = VOCAB // 2
PITCH = 136

_mesh = plsc.VectorSubcoreMesh(core_axis_name="c", subcore_axis_name="s")


@functools.partial(
    pl.kernel,
    mesh=_mesh,
    compiler_params=pltpu.CompilerParams(
        use_tc_tiling_on_sc=True, needs_layout_passes=False),
    out_type=jax.ShapeDtypeStruct((HIST, EMBED_DIM, BATCH), jnp.float32),
    scratch_types=[
        pltpu.VMEM((HIST, BPW), jnp.int32),
        pltpu.VMEM((BPW,), jnp.int32),
        pltpu.VMEM((BPW,), jnp.int32),
        pltpu.VMEM((BPW, PITCH), jnp.float32),
        pltpu.VMEM((BPW, PITCH), jnp.float32),
        pltpu.VMEM((EMBED_DIM, PITCH), jnp.float32),
        pltpu.VMEM((EMBED_DIM, PITCH), jnp.float32),
        pltpu.SemaphoreType.DMA,
        pltpu.SemaphoreType.DMA,
        pltpu.SemaphoreType.DMA,
        pltpu.SemaphoreType.DMA,
    ],
)
def _gather_kernel(table_p, idx_t, out, idx_v, p0, p1, gb0, gb1, ob0, ob1,
                   g0, g1, o0, o1):
    wid = lax.axis_index("s") * NC + lax.axis_index("c")
    b0 = wid * BPW
    pltpu.sync_copy(idx_t.at[:, pl.ds(b0, BPW)], idx_v)

    iota = lax.iota(jnp.int32, 16)
    rowm = [(iota + 16 * kb) * PITCH for kb in range(8)]

    def prep(h, pidx):
        for kb in range(8):
            x = idx_v[h, pl.ds(kb * 16, 16)]
            pidx[pl.ds(kb * 16, 16)] = x >> 1

    def gather(pidx, gb, sem):
        pltpu.async_copy(table_p.at[pidx], gb.at[:, pl.ds(0, 128)], sem)

    def wait_gather(pidx, gb, sem):
        pltpu.make_async_copy(table_p.at[pidx], gb.at[:, pl.ds(0, 128)], sem).wait()

    def repack(h, gb, ob):
        def kbody(kb, carry):
            xv = idx_v[h, pl.ds(kb * 16, 16)]
            offv = (xv & 1) << 6
            offs = [offv[kk] for kk in range(16)]
            vals = []
            for kk in range(16):
                row = kb * 16 + kk
                for jb in range(4):
                    vals.append(gb[row, pl.ds(offs[kk] + jb * 16, 16)])
            for kk in range(16):
                row = kb * 16 + kk
                kv = jnp.full((16,), row, jnp.int32)
                for jb in range(4):
                    plsc.store_scatter(ob, [iota + jb * 16, kv],
                                       vals[kk * 4 + jb])
            return carry

        lax.fori_loop(0, 8, kbody, 0)

    def out_issue(h, ob, sem):
        pltpu.async_copy(ob.at[:, pl.ds(0, BPW)], out.at[h, :, pl.ds(b0, BPW)], sem)

    def wait_out(h, ob, sem):
        pltpu.make_async_copy(
            ob.at[:, pl.ds(0, BPW)], out.at[h, :, pl.ds(b0, BPW)], sem).wait()

    prep(0, p0)
    gather(p0, gb0, g0)
    prep(1, p1)
    gather(p1, gb1, g1)

    wait_gather(p0, gb0, g0)
    repack(0, gb0, ob0)
    out_issue(0, ob0, o0)
    prep(2, p0)
    gather(p0, gb0, g0)
    wait_gather(p1, gb1, g1)
    repack(1, gb1, ob1)
    out_issue(1, ob1, o1)
    prep(3, p1)
    gather(p1, gb1, g1)

    def body(t, carry):
        h0 = 2 * t
        h1 = 2 * t + 1
        wait_gather(p0, gb0, g0)
        wait_out(h0 - 2, ob0, o0)
        repack(h0, gb0, ob0)
        out_issue(h0, ob0, o0)
        prep(h0 + 2, p0)
        gather(p0, gb0, g0)
        wait_gather(p1, gb1, g1)
        wait_out(h1 - 2, ob1, o1)
        repack(h1, gb1, ob1)
        out_issue(h1, ob1, o1)
        prep(h1 + 2, p1)
        gather(p1, gb1, g1)
        return carry

    lax.fori_loop(1, HIST // 2 - 1, body, 0)

    wait_gather(p0, gb0, g0)
    wait_out(46, ob0, o0)
    repack(48, gb0, ob0)
    out_issue(48, ob0, o0)
    wait_gather(p1, gb1, g1)
    wait_out(47, ob1, o1)
    repack(49, gb1, ob1)
    out_issue(49, ob1, o1)
    wait_out(48, ob0, o0)
    wait_out(49, ob1, o1)


def kernel(indices, table):
    idx_t = indices.T
    table_p = table.reshape(PAIRS, 128)
    out_phys = _gather_kernel(table_p, idx_t)
    return jnp.transpose(out_phys, (2, 0, 1))


# speedup vs baseline: 1.3901x; 1.3472x over previous
import functools

import jax
import jax.numpy as jnp
from jax import lax
from jax.experimental import pallas as pl
from jax.experimental.pallas import tpu as pltpu
from jax.experimental.pallas import tpu_sc as plsc

VOCAB = 100000
EMBED_DIM = 64
BATCH = 4096
HIST = 50

N = BATCH * HIST
NC = 2
NS = 16
NW = NC * NS
PER_W = N // NW
CHUNK = 800
NCHUNK = PER_W // CHUNK

_mesh = plsc.VectorSubcoreMesh(core_axis_name="c", subcore_axis_name="s")


@functools.partial(
    pl.kernel,
    mesh=_mesh,
    compiler_params=pltpu.CompilerParams(use_tc_tiling_on_sc=False),
    out_type=jax.ShapeDtypeStruct((N, EMBED_DIM), jnp.float32),
    scratch_types=[
        pltpu.VMEM((NCHUNK, CHUNK), jnp.int32),
        pltpu.VMEM((CHUNK, EMBED_DIM), jnp.float32),
        pltpu.VMEM((CHUNK, EMBED_DIM), jnp.float32),
        pltpu.SemaphoreType.DMA,
        pltpu.SemaphoreType.DMA,
        pltpu.SemaphoreType.DMA,
        pltpu.SemaphoreType.DMA,
    ],
)
def _gather_kernel(table_hbm, idx_hbm, out_hbm, idx_v, rows0, rows1, g0, g1, s0, s1):
    wid = lax.axis_index("s") * NC + lax.axis_index("c")
    base = wid * PER_W
    pltpu.sync_copy(idx_hbm.at[wid], idx_v)

    bufs = (rows0, rows1)
    gsems = (g0, g1)
    ssems = (s0, s1)
    gath = [None] * NCHUNK
    scat = [None] * NCHUNK

    for ci in range(NCHUNK):
        b = ci % 2
        if ci >= 2:
            scat[ci - 2].wait()
        gath[ci] = pltpu.async_copy(table_hbm.at[idx_v.at[ci]], bufs[b], gsems[b])
        if ci >= 1:
            gath[ci - 1].wait()
            scat[ci - 1] = pltpu.async_copy(
                bufs[1 - b],
                out_hbm.at[pl.ds(base + (ci - 1) * CHUNK, CHUNK)],
                ssems[1 - b],
            )
    gath[NCHUNK - 1].wait()
    scat[NCHUNK - 1] = pltpu.async_copy(
        bufs[(NCHUNK - 1) % 2],
        out_hbm.at[pl.ds(base + (NCHUNK - 1) * CHUNK, CHUNK)],
        ssems[(NCHUNK - 1) % 2],
    )
    scat[NCHUNK - 2].wait()
    scat[NCHUNK - 1].wait()


def kernel(indices, table):
    idx3 = indices.reshape(NW, NCHUNK, CHUNK)
    out = _gather_kernel(table, idx3)
    return out.reshape(BATCH, HIST, EMBED_DIM)
